# Initial kernel scaffold; baseline (speedup 1.0000x reference)
import jax
import jax.numpy as jnp
from jax.experimental import pallas as pl


def kernel(x, edge_index, edge_weight, p, W_ih, W_hh, b_ih, b_hh, W0, W_lin, b_lin):
    raise NotImplementedError("write your pallas kernel here")


# speedup vs baseline: 13.0272x; 13.0272x over previous
import functools

import jax
import jax.numpy as jnp
from jax import lax
from jax.experimental import pallas as pl
from jax.experimental.pallas import tpu as pltpu
from jax.experimental.pallas import tpu_sc as plsc

N = 10000
NP = 10240
F = 128
E = 320000
NC = 2
NS = 16
TILE_E = E // (NC * NS)
CH = 80
NCHUNK = TILE_E // CH
ROWS_PER_TILE = 624
TAIL_START = ROWS_PER_TILE * NS
TAIL = N - TAIL_START


def _sc_mesh():
    return plsc.VectorSubcoreMesh(
        core_axis_name="c", subcore_axis_name="s", num_cores=NC, num_subcores=NS
    )


def _deg_body(dst_hbm, ew_hbm, zeros_hbm, deg_hbm, dstv, ewv, zb, degsp):
    c = lax.axis_index("c")
    s = lax.axis_index("s")
    wid = c * NS + s

    @pl.when(s == 0)
    def _():
        pltpu.sync_copy(zeros_hbm, zb)
        pltpu.sync_copy(zb, degsp)

    plsc.subcore_barrier()

    def chunk(j, carry):
        base = wid * TILE_E + j * CH
        pltpu.sync_copy(dst_hbm.at[pl.ds(base, CH)], dstv)
        pltpu.sync_copy(ew_hbm.at[pl.ds(base, CH)], ewv)
        pltpu.sync_copy(ewv, degsp.at[dstv], add=True)
        return carry

    lax.fori_loop(0, NCHUNK, chunk, 0)
    plsc.subcore_barrier()

    @pl.when(s == 0)
    def _():
        pltpu.sync_copy(degsp, zb)
        pltpu.sync_copy(zb, deg_hbm.at[c])


def _deg_call(dst, edge_weight, zeros1):
    return pl.kernel(
        _deg_body,
        out_type=jax.ShapeDtypeStruct((NC, N), jnp.float32),
        mesh=_sc_mesh(),
        compiler_params=pltpu.CompilerParams(needs_layout_passes=False),
        scratch_types=[
            pltpu.VMEM((CH,), jnp.int32),
            pltpu.VMEM((CH,), jnp.float32),
            pltpu.VMEM((N,), jnp.float32),
            pltpu.VMEM_SHARED((N,), jnp.float32),
        ],
    )(dst, edge_weight, zeros1)


def _pre_body(xT_ref, x_ref, p_ref, wihT_ref, whhT_ref, bih_ref, bhh_ref,
              w0_ref, xw_ref, sc_ref, xt_ref, row_buf, srow_buf):
    pv = p_ref[...]
    pn11 = lax.rsqrt(jnp.sum(pv * pv, axis=1, keepdims=True))
    iota = lax.broadcasted_iota(jnp.int32, (1, NP), 1)
    neg_inf = jnp.float32(-jnp.inf)
    raw = jnp.dot(pv, xT_ref[...],
                  preferred_element_type=jnp.float32) * pn11
    sc_ref[...] = jnp.where(iota < N, raw, neg_inf)

    def step(i, carry):
        scv = sc_ref[...]
        m11 = jnp.max(scv, axis=1, keepdims=True)
        idx11 = jnp.min(jnp.where(scv == m11, iota, jnp.int32(NP)),
                        axis=1, keepdims=True)
        idx = idx11[0, 0]
        pltpu.sync_copy(x_ref.at[pl.ds(idx, 1), :], row_buf)
        srow_buf[...] = row_buf[...] * jnp.tanh(m11)
        pltpu.sync_copy(srow_buf, xt_ref.at[pl.ds(i, 1), :])
        sc_ref[...] = jnp.where(iota == idx11, neg_inf, scv)
        return carry

    lax.fori_loop(0, F, step, 0)

    xt = xt_ref[...]
    gi = jnp.dot(xt, wihT_ref[...], preferred_element_type=jnp.float32) + bih_ref[...]
    gh = jnp.dot(w0_ref[...], whhT_ref[...], preferred_element_type=jnp.float32) + bhh_ref[...]
    r = jax.nn.sigmoid(gi[:, :F] + gh[:, :F])
    z = jax.nn.sigmoid(gi[:, F:2 * F] + gh[:, F:2 * F])
    cand = jnp.tanh(gi[:, 2 * F:] + r * gh[:, 2 * F:])
    w_ev = (1.0 - z) * cand + z * w0_ref[...]
    xw_ref[...] = jnp.dot(x_ref[...], w_ev, preferred_element_type=jnp.float32)


def _pre_call(xT, x, p2, wihT, whhT, bih2, bhh2, W0):
    return pl.pallas_call(
        _pre_body,
        out_shape=jax.ShapeDtypeStruct((N, F), jnp.float32),
        scratch_shapes=[
            pltpu.VMEM((1, NP), jnp.float32),
            pltpu.VMEM((F, F), jnp.float32),
            pltpu.VMEM((1, F), jnp.float32),
            pltpu.VMEM((1, F), jnp.float32),
        ],
    )(xT, x, p2, wihT, whhT, bih2, bhh2, W0)


def _agg_body(src_hbm, dst_hbm, ew_hbm, xw_hbm, dinv_hbm, zeros_hbm, outp_hbm,
              dinv_v, rows, srcv, dstv, ewv, nrm, outsp, gsem):
    c = lax.axis_index("c")
    s = lax.axis_index("s")
    wid = c * NS + s

    pltpu.sync_copy(dinv_hbm, dinv_v)

    pltpu.sync_copy(zeros_hbm, rows)
    for k in range(7):
        pltpu.sync_copy(rows, outsp.at[pl.ds(s * ROWS_PER_TILE + k * CH, CH)])
    pltpu.sync_copy(rows.at[pl.ds(0, 64)],
                    outsp.at[pl.ds(s * ROWS_PER_TILE + 560, 64)])

    @pl.when(s == NS - 1)
    def _():
        pltpu.sync_copy(rows.at[pl.ds(0, TAIL)],
                        outsp.at[pl.ds(TAIL_START, TAIL)])

    plsc.subcore_barrier()

    def chunk(j, carry):
        base = wid * TILE_E + j * CH
        pltpu.sync_copy(src_hbm.at[pl.ds(base, CH)], srcv)
        pltpu.sync_copy(dst_hbm.at[pl.ds(base, CH)], dstv)
        pltpu.sync_copy(ew_hbm.at[pl.ds(base, CH)], ewv)
        pltpu.async_copy(xw_hbm.at[srcv], rows, gsem).wait()

        def nstep(t, carry2):
            sl = pl.ds(t * 16, 16)
            a = plsc.load_gather(dinv_v, [srcv[sl]])
            b = plsc.load_gather(dinv_v, [dstv[sl]])
            nrm[sl] = a * ewv[sl] * b
            return carry2

        lax.fori_loop(0, CH // 16, nstep, 0)

        def sstep(g, carry2):
            nvec = nrm[pl.ds(g * 16, 16)]
            for i2 in range(16):
                nv = nvec[i2]
                e = g * 16 + i2
                for k in range(F // 16):
                    sl = pl.ds(k * 16, 16)
                    rows[e, sl] = rows[e, sl] * nv
            return carry2

        lax.fori_loop(0, CH // 16, sstep, 0)
        pltpu.sync_copy(rows, outsp.at[dstv], add=True)
        return carry

    lax.fori_loop(0, NCHUNK, chunk, 0)
    plsc.subcore_barrier()

    for k in range(7):
        off = s * ROWS_PER_TILE + k * CH
        pltpu.sync_copy(outsp.at[pl.ds(off, CH)], rows)
        pltpu.sync_copy(rows, outp_hbm.at[c, pl.ds(off, CH)])
    off64 = s * ROWS_PER_TILE + 560
    pltpu.sync_copy(outsp.at[pl.ds(off64, 64)], rows.at[pl.ds(0, 64)])
    pltpu.sync_copy(rows.at[pl.ds(0, 64)], outp_hbm.at[c, pl.ds(off64, 64)])

    @pl.when(s == NS - 1)
    def _():
        pltpu.sync_copy(outsp.at[pl.ds(TAIL_START, TAIL)], rows.at[pl.ds(0, TAIL)])
        pltpu.sync_copy(rows.at[pl.ds(0, TAIL)],
                        outp_hbm.at[c, pl.ds(TAIL_START, TAIL)])


def _agg_call(src, dst, edge_weight, xw, dinv1, zeros2):
    return pl.kernel(
        _agg_body,
        out_type=jax.ShapeDtypeStruct((NC, N, F), jnp.float32),
        mesh=_sc_mesh(),
        compiler_params=pltpu.CompilerParams(needs_layout_passes=False),
        scratch_types=[
            pltpu.VMEM((N,), jnp.float32),
            pltpu.VMEM((CH, F), jnp.float32),
            pltpu.VMEM((CH,), jnp.int32),
            pltpu.VMEM((CH,), jnp.int32),
            pltpu.VMEM((CH,), jnp.float32),
            pltpu.VMEM((CH,), jnp.float32),
            pltpu.VMEM_SHARED((N, F), jnp.float32),
            pltpu.SemaphoreType.DMA,
        ],
    )(src, dst, edge_weight, xw, dinv1, zeros2)


def _dinv_body(deg_ref, o_ref):
    o_ref[...] = lax.rsqrt(deg_ref[0] + deg_ref[1] + 1.0)


def _dinv_call(deg3):
    return pl.pallas_call(
        _dinv_body,
        out_shape=jax.ShapeDtypeStruct((80, 125), jnp.float32),
    )(deg3)


def _post_body(parts_ref, dinv_ref, xw_ref, wlinT_ref, blin_ref, y_ref):
    dv = dinv_ref[...]
    o = parts_ref[0] + parts_ref[1] + dv * dv * xw_ref[...]
    h = jnp.maximum(o, 0.0)
    y_ref[...] = jnp.dot(h, wlinT_ref[...],
                         preferred_element_type=jnp.float32) + blin_ref[...]


def _post_call(outp, dinv2, xw, wlinT, blin2):
    return pl.pallas_call(
        _post_body,
        out_shape=jax.ShapeDtypeStruct((N, 1), jnp.float32),
    )(outp, dinv2, xw, wlinT, blin2)


def kernel(x, edge_index, edge_weight, p, W_ih, W_hh, b_ih, b_hh, W0, W_lin, b_lin):
    xT = jnp.pad(x.T, ((0, 0), (0, NP - N)))
    p2 = p.reshape(1, F)
    wihT = W_ih.T
    whhT = W_hh.T
    bih2 = b_ih.reshape(1, 3 * F)
    bhh2 = b_hh.reshape(1, 3 * F)
    zeros1 = jnp.zeros((N,), jnp.float32)
    zeros2 = jnp.zeros((CH, F), jnp.float32)
    src = edge_index[0]
    dst = edge_index[1]

    deg_parts = _deg_call(dst, edge_weight, zeros1)
    xw = _pre_call(xT, x, p2, wihT, whhT, bih2, bhh2, W0)
    dinv = _dinv_call(deg_parts.reshape(NC, 80, 125))
    outp = _agg_call(src, dst, edge_weight, xw, dinv.reshape(N), zeros2)
    y = _post_call(outp, dinv.reshape(N, 1), xw, W_lin.T, b_lin.reshape(1, 1))
    return y


# speedup vs baseline: 20.8378x; 1.5996x over previous
import functools

import jax
import jax.numpy as jnp
from jax import lax
from jax.experimental import pallas as pl
from jax.experimental.pallas import tpu as pltpu
from jax.experimental.pallas import tpu_sc as plsc

N = 10000
NP = 10240
F = 128
E = 320000
NC = 2
NS = 16
TILE_E = E // (NC * NS)
CH = 80
NCHUNK = TILE_E // CH
ROWS_PER_TILE = 624
TAIL_START = ROWS_PER_TILE * NS
TAIL = N - TAIL_START


def _sc_mesh():
    return plsc.VectorSubcoreMesh(
        core_axis_name="c", subcore_axis_name="s", num_cores=NC, num_subcores=NS
    )


def _deg_body(dst_hbm, ew_hbm, zeros_hbm, deg_hbm, dstv, ewv, zb, degsp):
    c = lax.axis_index("c")
    s = lax.axis_index("s")
    wid = c * NS + s

    @pl.when(s == 0)
    def _():
        pltpu.sync_copy(zeros_hbm, zb)
        pltpu.sync_copy(zb, degsp)

    plsc.subcore_barrier()

    def chunk(j, carry):
        base = wid * TILE_E + j * CH
        pltpu.sync_copy(dst_hbm.at[pl.ds(base, CH)], dstv)
        pltpu.sync_copy(ew_hbm.at[pl.ds(base, CH)], ewv)
        pltpu.sync_copy(ewv, degsp.at[dstv], add=True)
        return carry

    lax.fori_loop(0, NCHUNK, chunk, 0)
    plsc.subcore_barrier()

    @pl.when(s == 0)
    def _():
        pltpu.sync_copy(degsp, zb)
        pltpu.sync_copy(zb, deg_hbm.at[c])


def _deg_call(dst, edge_weight, zeros1):
    return pl.kernel(
        _deg_body,
        out_type=jax.ShapeDtypeStruct((NC, N), jnp.float32),
        mesh=_sc_mesh(),
        compiler_params=pltpu.CompilerParams(needs_layout_passes=False),
        scratch_types=[
            pltpu.VMEM((CH,), jnp.int32),
            pltpu.VMEM((CH,), jnp.float32),
            pltpu.VMEM((N,), jnp.float32),
            pltpu.VMEM_SHARED((N,), jnp.float32),
        ],
    )(dst, edge_weight, zeros1)


def _pre_body(xT_ref, x_ref, p_ref, wihT_ref, whhT_ref, bih_ref, bhh_ref,
              w0_ref, xw_ref, sc_ref, xt_ref, row_buf, srow_buf):
    pv = p_ref[...]
    pn11 = lax.rsqrt(jnp.sum(pv * pv, axis=1, keepdims=True))
    iota = lax.broadcasted_iota(jnp.int32, (1, NP), 1)
    neg_inf = jnp.float32(-jnp.inf)
    raw = jnp.dot(pv, xT_ref[...],
                  preferred_element_type=jnp.float32) * pn11
    sc_ref[...] = jnp.where(iota < N, raw, neg_inf)

    def step(i, carry):
        scv = sc_ref[...]
        m11 = jnp.max(scv, axis=1, keepdims=True)
        idx11 = jnp.min(jnp.where(scv == m11, iota, jnp.int32(NP)),
                        axis=1, keepdims=True)
        idx = idx11[0, 0]
        pltpu.sync_copy(x_ref.at[pl.ds(idx, 1), :], row_buf)
        srow_buf[...] = row_buf[...] * jnp.tanh(m11)
        pltpu.sync_copy(srow_buf, xt_ref.at[pl.ds(i, 1), :])
        sc_ref[...] = jnp.where(iota == idx11, neg_inf, scv)
        return carry

    lax.fori_loop(0, F, step, 0)

    xt = xt_ref[...]
    gi = jnp.dot(xt, wihT_ref[...], preferred_element_type=jnp.float32) + bih_ref[...]
    gh = jnp.dot(w0_ref[...], whhT_ref[...], preferred_element_type=jnp.float32) + bhh_ref[...]
    r = jax.nn.sigmoid(gi[:, :F] + gh[:, :F])
    z = jax.nn.sigmoid(gi[:, F:2 * F] + gh[:, F:2 * F])
    cand = jnp.tanh(gi[:, 2 * F:] + r * gh[:, 2 * F:])
    w_ev = (1.0 - z) * cand + z * w0_ref[...]
    xw_ref[...] = jnp.dot(x_ref[...], w_ev, preferred_element_type=jnp.float32)


def _pre_call(xT, x, p2, wihT, whhT, bih2, bhh2, W0):
    return pl.pallas_call(
        _pre_body,
        out_shape=jax.ShapeDtypeStruct((N, F), jnp.float32),
        scratch_shapes=[
            pltpu.VMEM((1, NP), jnp.float32),
            pltpu.VMEM((F, F), jnp.float32),
            pltpu.VMEM((1, F), jnp.float32),
            pltpu.VMEM((1, F), jnp.float32),
        ],
    )(xT, x, p2, wihT, whhT, bih2, bhh2, W0)


UN = 3


def _agg_body(src_hbm, dst_hbm, ew_hbm, xw_hbm, dinv_hbm, zeros_hbm, outp_hbm,
              dinv_v, nrm,
              src0, src1, src2, dst0, dst1, dst2, ew0, ew1, ew2,
              rows0, rows1, rows2,
              is0, is1, is2, gs0, gs1, gs2, outsp):
    c = lax.axis_index("c")
    s = lax.axis_index("s")
    wid = c * NS + s
    ebase = wid * TILE_E
    srcv = [src0, src1, src2]
    dstv = [dst0, dst1, dst2]
    ewv = [ew0, ew1, ew2]
    rows = [rows0, rows1, rows2]
    isem = [is0, is1, is2]
    gsem = [gs0, gs1, gs2]

    def issue_idx(b, cid):
        base = ebase + cid * CH
        pltpu.async_copy(src_hbm.at[pl.ds(base, CH)], srcv[b], isem[b])
        pltpu.async_copy(dst_hbm.at[pl.ds(base, CH)], dstv[b], isem[b])
        pltpu.async_copy(ew_hbm.at[pl.ds(base, CH)], ewv[b], isem[b])

    def wait_idx(b):
        pltpu.make_async_copy(src_hbm.at[pl.ds(0, CH)], srcv[b], isem[b]).wait()
        pltpu.make_async_copy(dst_hbm.at[pl.ds(0, CH)], dstv[b], isem[b]).wait()
        pltpu.make_async_copy(ew_hbm.at[pl.ds(0, CH)], ewv[b], isem[b]).wait()

    pltpu.sync_copy(dinv_hbm, dinv_v)

    pltpu.sync_copy(zeros_hbm, rows0)
    for k in range(7):
        pltpu.sync_copy(rows0, outsp.at[pl.ds(s * ROWS_PER_TILE + k * CH, CH)])
    pltpu.sync_copy(rows0.at[pl.ds(0, 64)],
                    outsp.at[pl.ds(s * ROWS_PER_TILE + 560, 64)])

    @pl.when(s == NS - 1)
    def _():
        pltpu.sync_copy(rows0.at[pl.ds(0, TAIL)],
                        outsp.at[pl.ds(TAIL_START, TAIL)])

    plsc.subcore_barrier()

    for b in range(UN):
        issue_idx(b, b)

    NG = (NCHUNK + UN - 1) // UN

    def grp(g, carry):
        gds = {}
        for b in range(UN):
            cid = g * UN + b

            @pl.when(cid < NCHUNK)
            def _(b=b):
                wait_idx(b)
                gds[b] = pltpu.async_copy(xw_hbm.at[srcv[b]], rows[b], gsem[b])
        for b in range(UN):
            cid = g * UN + b

            @pl.when(cid < NCHUNK)
            def _(b=b, cid=cid):
                gds[b].wait()

                def nstep(t, carry2):
                    sl = pl.ds(t * 16, 16)
                    a = plsc.load_gather(dinv_v, [srcv[b][sl]])
                    bb = plsc.load_gather(dinv_v, [dstv[b][sl]])
                    nrm[sl] = a * ewv[b][sl] * bb
                    return carry2

                lax.fori_loop(0, CH // 16, nstep, 0)

                def sstep(gg, carry2):
                    nvec = nrm[pl.ds(gg * 16, 16)]
                    for i2 in range(16):
                        nv = nvec[i2]
                        e = gg * 16 + i2
                        for k in range(F // 16):
                            sl = pl.ds(k * 16, 16)
                            rows[b][e, sl] = rows[b][e, sl] * nv
                    return carry2

                lax.fori_loop(0, CH // 16, sstep, 0)
                pltpu.sync_copy(rows[b], outsp.at[dstv[b]], add=True)

                @pl.when(cid + UN < NCHUNK)
                def _():
                    issue_idx(b, cid + UN)
        return carry

    lax.fori_loop(0, NG, grp, 0)
    plsc.subcore_barrier()

    for k in range(7):
        off = s * ROWS_PER_TILE + k * CH
        pltpu.sync_copy(outsp.at[pl.ds(off, CH)], rows0)
        pltpu.sync_copy(rows0, outp_hbm.at[c, pl.ds(off, CH)])
    off64 = s * ROWS_PER_TILE + 560
    pltpu.sync_copy(outsp.at[pl.ds(off64, 64)], rows0.at[pl.ds(0, 64)])
    pltpu.sync_copy(rows0.at[pl.ds(0, 64)], outp_hbm.at[c, pl.ds(off64, 64)])

    @pl.when(s == NS - 1)
    def _():
        pltpu.sync_copy(outsp.at[pl.ds(TAIL_START, TAIL)],
                        rows0.at[pl.ds(0, TAIL)])
        pltpu.sync_copy(rows0.at[pl.ds(0, TAIL)],
                        outp_hbm.at[c, pl.ds(TAIL_START, TAIL)])


def _agg_call(src, dst, edge_weight, xw, dinv1, zeros2):
    return pl.kernel(
        _agg_body,
        out_type=jax.ShapeDtypeStruct((NC, N, F), jnp.float32),
        mesh=_sc_mesh(),
        compiler_params=pltpu.CompilerParams(needs_layout_passes=False),
        scratch_types=[
            pltpu.VMEM((N,), jnp.float32),
            pltpu.VMEM((CH,), jnp.float32),
            pltpu.VMEM((CH,), jnp.int32),
            pltpu.VMEM((CH,), jnp.int32),
            pltpu.VMEM((CH,), jnp.int32),
            pltpu.VMEM((CH,), jnp.int32),
            pltpu.VMEM((CH,), jnp.int32),
            pltpu.VMEM((CH,), jnp.int32),
            pltpu.VMEM((CH,), jnp.float32),
            pltpu.VMEM((CH,), jnp.float32),
            pltpu.VMEM((CH,), jnp.float32),
            pltpu.VMEM((CH, F), jnp.float32),
            pltpu.VMEM((CH, F), jnp.float32),
            pltpu.VMEM((CH, F), jnp.float32),
            pltpu.SemaphoreType.DMA,
            pltpu.SemaphoreType.DMA,
            pltpu.SemaphoreType.DMA,
            pltpu.SemaphoreType.DMA,
            pltpu.SemaphoreType.DMA,
            pltpu.SemaphoreType.DMA,
            pltpu.VMEM_SHARED((N, F), jnp.float32),
        ],
    )(src, dst, edge_weight, xw, dinv1, zeros2)


def _dinv_body(deg_ref, o_ref):
    o_ref[...] = lax.rsqrt(deg_ref[0] + deg_ref[1] + 1.0)


def _dinv_call(deg3):
    return pl.pallas_call(
        _dinv_body,
        out_shape=jax.ShapeDtypeStruct((80, 125), jnp.float32),
    )(deg3)


def _post_body(parts_ref, dinv_ref, xw_ref, wlinT_ref, blin_ref, y_ref):
    dv = dinv_ref[...]
    o = parts_ref[0] + parts_ref[1] + dv * dv * xw_ref[...]
    h = jnp.maximum(o, 0.0)
    y_ref[...] = jnp.dot(h, wlinT_ref[...],
                         preferred_element_type=jnp.float32) + blin_ref[...]


def _post_call(outp, dinv2, xw, wlinT, blin2):
    return pl.pallas_call(
        _post_body,
        out_shape=jax.ShapeDtypeStruct((N, 1), jnp.float32),
    )(outp, dinv2, xw, wlinT, blin2)


def kernel(x, edge_index, edge_weight, p, W_ih, W_hh, b_ih, b_hh, W0, W_lin, b_lin):
    xT = jnp.pad(x.T, ((0, 0), (0, NP - N)))
    p2 = p.reshape(1, F)
    wihT = W_ih.T
    whhT = W_hh.T
    bih2 = b_ih.reshape(1, 3 * F)
    bhh2 = b_hh.reshape(1, 3 * F)
    zeros1 = jnp.zeros((N,), jnp.float32)
    zeros2 = jnp.zeros((CH, F), jnp.float32)
    src = edge_index[0]
    dst = edge_index[1]

    deg_parts = _deg_call(dst, edge_weight, zeros1)
    xw = _pre_call(xT, x, p2, wihT, whhT, bih2, bhh2, W0)
    dinv = _dinv_call(deg_parts.reshape(NC, 80, 125))
    outp = _agg_call(src, dst, edge_weight, xw, dinv.reshape(N), zeros2)
    y = _post_call(outp, dinv.reshape(N, 1), xw, W_lin.T, b_lin.reshape(1, 1))
    return y


# speedup vs baseline: 24.2433x; 1.1634x over previous
import functools

import jax
import jax.numpy as jnp
from jax import lax
from jax.experimental import pallas as pl
from jax.experimental.pallas import tpu as pltpu
from jax.experimental.pallas import tpu_sc as plsc

N = 10000
NP = 10240
F = 128
E = 320000
NC = 2
NS = 16
TILE_E = E // (NC * NS)
CH = 80
NCHUNK = TILE_E // CH
ROWS_PER_TILE = 624
TAIL_START = ROWS_PER_TILE * NS
TAIL = N - TAIL_START


def _sc_mesh():
    return plsc.VectorSubcoreMesh(
        core_axis_name="c", subcore_axis_name="s", num_cores=NC, num_subcores=NS
    )


DUN = 5


def _deg_body(dst_hbm, ew_hbm, zeros_hbm, deg_hbm,
              dst0, dst1, dst2, dst3, dst4, ew0, ew1, ew2, ew3, ew4,
              zb, sm0, sm1, sm2, sm3, sm4, degsp):
    c = lax.axis_index("c")
    s = lax.axis_index("s")
    wid = c * NS + s
    ebase = wid * TILE_E
    dstv = [dst0, dst1, dst2, dst3, dst4]
    ewv = [ew0, ew1, ew2, ew3, ew4]
    sems = [sm0, sm1, sm2, sm3, sm4]

    def issue(b, cid):
        base = ebase + cid * CH
        pltpu.async_copy(dst_hbm.at[pl.ds(base, CH)], dstv[b], sems[b])
        pltpu.async_copy(ew_hbm.at[pl.ds(base, CH)], ewv[b], sems[b])

    def wait(b):
        pltpu.make_async_copy(dst_hbm.at[pl.ds(0, CH)], dstv[b], sems[b]).wait()
        pltpu.make_async_copy(ew_hbm.at[pl.ds(0, CH)], ewv[b], sems[b]).wait()

    @pl.when(s == 0)
    def _():
        pltpu.sync_copy(zeros_hbm, zb)
        pltpu.sync_copy(zb, degsp)

    plsc.subcore_barrier()

    for b in range(DUN):
        issue(b, b)

    def grp(g, carry):
        for b in range(DUN):
            cid = g * DUN + b
            wait(b)
            pltpu.sync_copy(ewv[b], degsp.at[dstv[b]], add=True)

            @pl.when(cid + DUN < NCHUNK)
            def _(b=b, cid=cid):
                issue(b, cid + DUN)
        return carry

    lax.fori_loop(0, NCHUNK // DUN, grp, 0)
    plsc.subcore_barrier()

    @pl.when(s == 0)
    def _():
        pltpu.sync_copy(degsp, zb)
        pltpu.sync_copy(zb, deg_hbm.at[c])


def _deg_call(dst, edge_weight, zeros1):
    return pl.kernel(
        _deg_body,
        out_type=jax.ShapeDtypeStruct((NC, N), jnp.float32),
        mesh=_sc_mesh(),
        compiler_params=pltpu.CompilerParams(needs_layout_passes=False),
        scratch_types=(
            [pltpu.VMEM((CH,), jnp.int32)] * 5
            + [pltpu.VMEM((CH,), jnp.float32)] * 5
            + [pltpu.VMEM((N,), jnp.float32)]
            + [pltpu.SemaphoreType.DMA] * 5
            + [pltpu.VMEM_SHARED((N,), jnp.float32)]
        ),
    )(dst, edge_weight, zeros1)


def _pre_body(xT_ref, x_ref, p_ref, wihT_ref, whhT_ref, bih_ref, bhh_ref,
              w0_ref, xw_ref, sc_ref, xt_ref, row_buf, srow_buf):
    pv = p_ref[...]
    pn11 = lax.rsqrt(jnp.sum(pv * pv, axis=1, keepdims=True))
    iota = lax.broadcasted_iota(jnp.int32, (1, NP), 1)
    neg_inf = jnp.float32(-jnp.inf)
    raw = jnp.dot(pv, xT_ref[...],
                  preferred_element_type=jnp.float32) * pn11
    sc_ref[...] = jnp.where(iota < N, raw, neg_inf)

    def step(i, carry):
        scv = sc_ref[...]
        m11 = jnp.max(scv, axis=1, keepdims=True)
        idx11 = jnp.min(jnp.where(scv == m11, iota, jnp.int32(NP)),
                        axis=1, keepdims=True)
        idx = idx11[0, 0]
        pltpu.sync_copy(x_ref.at[pl.ds(idx, 1), :], row_buf)
        srow_buf[...] = row_buf[...] * jnp.tanh(m11)
        pltpu.sync_copy(srow_buf, xt_ref.at[pl.ds(i, 1), :])
        sc_ref[...] = jnp.where(iota == idx11, neg_inf, scv)
        return carry

    lax.fori_loop(0, F, step, 0)

    xt = xt_ref[...]
    gi = jnp.dot(xt, wihT_ref[...], preferred_element_type=jnp.float32) + bih_ref[...]
    gh = jnp.dot(w0_ref[...], whhT_ref[...], preferred_element_type=jnp.float32) + bhh_ref[...]
    r = jax.nn.sigmoid(gi[:, :F] + gh[:, :F])
    z = jax.nn.sigmoid(gi[:, F:2 * F] + gh[:, F:2 * F])
    cand = jnp.tanh(gi[:, 2 * F:] + r * gh[:, 2 * F:])
    w_ev = (1.0 - z) * cand + z * w0_ref[...]
    xw_ref[...] = jnp.dot(x_ref[...], w_ev, preferred_element_type=jnp.float32)


def _pre_call(xT, x, p2, wihT, whhT, bih2, bhh2, W0):
    return pl.pallas_call(
        _pre_body,
        out_shape=jax.ShapeDtypeStruct((N, F), jnp.float32),
        scratch_shapes=[
            pltpu.VMEM((1, NP), jnp.float32),
            pltpu.VMEM((F, F), jnp.float32),
            pltpu.VMEM((1, F), jnp.float32),
            pltpu.VMEM((1, F), jnp.float32),
        ],
    )(xT, x, p2, wihT, whhT, bih2, bhh2, W0)


UN = 3


def _agg_body(src_hbm, dst_hbm, ew_hbm, xw_hbm, dinv_hbm, zeros_hbm, outp_hbm,
              dinv_v, nrm,
              src0, src1, src2, dst0, dst1, dst2, ew0, ew1, ew2,
              rows0, rows1, rows2,
              is0, is1, is2, gs0, gs1, gs2, outsp):
    c = lax.axis_index("c")
    s = lax.axis_index("s")
    wid = c * NS + s
    ebase = wid * TILE_E
    srcv = [src0, src1, src2]
    dstv = [dst0, dst1, dst2]
    ewv = [ew0, ew1, ew2]
    rows = [rows0, rows1, rows2]
    isem = [is0, is1, is2]
    gsem = [gs0, gs1, gs2]

    def issue_idx(b, cid):
        base = ebase + cid * CH
        pltpu.async_copy(src_hbm.at[pl.ds(base, CH)], srcv[b], isem[b])
        pltpu.async_copy(dst_hbm.at[pl.ds(base, CH)], dstv[b], isem[b])
        pltpu.async_copy(ew_hbm.at[pl.ds(base, CH)], ewv[b], isem[b])

    def wait_idx(b):
        pltpu.make_async_copy(src_hbm.at[pl.ds(0, CH)], srcv[b], isem[b]).wait()
        pltpu.make_async_copy(dst_hbm.at[pl.ds(0, CH)], dstv[b], isem[b]).wait()
        pltpu.make_async_copy(ew_hbm.at[pl.ds(0, CH)], ewv[b], isem[b]).wait()

    pltpu.sync_copy(dinv_hbm, dinv_v)

    pltpu.sync_copy(zeros_hbm, rows0)
    for k in range(7):
        pltpu.sync_copy(rows0, outsp.at[pl.ds(s * ROWS_PER_TILE + k * CH, CH)])
    pltpu.sync_copy(rows0.at[pl.ds(0, 64)],
                    outsp.at[pl.ds(s * ROWS_PER_TILE + 560, 64)])

    @pl.when(s == NS - 1)
    def _():
        pltpu.sync_copy(rows0.at[pl.ds(0, TAIL)],
                        outsp.at[pl.ds(TAIL_START, TAIL)])

    plsc.subcore_barrier()

    for b in range(UN):
        issue_idx(b, b)

    NG = (NCHUNK + UN - 1) // UN

    def grp(g, carry):
        gds = {}
        for b in range(UN):
            cid = g * UN + b

            @pl.when(cid < NCHUNK)
            def _(b=b):
                wait_idx(b)
                gds[b] = pltpu.async_copy(xw_hbm.at[srcv[b]], rows[b], gsem[b])
        for b in range(UN):
            cid = g * UN + b

            @pl.when(cid < NCHUNK)
            def _(b=b, cid=cid):
                gds[b].wait()

                def nstep(t, carry2):
                    sl = pl.ds(t * 16, 16)
                    a = plsc.load_gather(dinv_v, [srcv[b][sl]])
                    bb = plsc.load_gather(dinv_v, [dstv[b][sl]])
                    nrm[sl] = a * ewv[b][sl] * bb
                    return carry2

                lax.fori_loop(0, CH // 16, nstep, 0)

                def sstep(gg, carry2):
                    nvec = nrm[pl.ds(gg * 16, 16)]
                    for i2 in range(16):
                        nv = nvec[i2]
                        e = gg * 16 + i2
                        for k in range(F // 16):
                            sl = pl.ds(k * 16, 16)
                            rows[b][e, sl] = rows[b][e, sl] * nv
                    return carry2

                lax.fori_loop(0, CH // 16, sstep, 0)
                pltpu.sync_copy(rows[b], outsp.at[dstv[b]], add=True)

                @pl.when(cid + UN < NCHUNK)
                def _():
                    issue_idx(b, cid + UN)
        return carry

    lax.fori_loop(0, NG, grp, 0)
    plsc.subcore_barrier()

    for k in range(7):
        off = s * ROWS_PER_TILE + k * CH
        pltpu.sync_copy(outsp.at[pl.ds(off, CH)], rows0)
        pltpu.sync_copy(rows0, outp_hbm.at[c, pl.ds(off, CH)])
    off64 = s * ROWS_PER_TILE + 560
    pltpu.sync_copy(outsp.at[pl.ds(off64, 64)], rows0.at[pl.ds(0, 64)])
    pltpu.sync_copy(rows0.at[pl.ds(0, 64)], outp_hbm.at[c, pl.ds(off64, 64)])

    @pl.when(s == NS - 1)
    def _():
        pltpu.sync_copy(outsp.at[pl.ds(TAIL_START, TAIL)],
                        rows0.at[pl.ds(0, TAIL)])
        pltpu.sync_copy(rows0.at[pl.ds(0, TAIL)],
                        outp_hbm.at[c, pl.ds(TAIL_START, TAIL)])


def _agg_call(src, dst, edge_weight, xw, dinv1, zeros2):
    return pl.kernel(
        _agg_body,
        out_type=jax.ShapeDtypeStruct((NC, N, F), jnp.float32),
        mesh=_sc_mesh(),
        compiler_params=pltpu.CompilerParams(needs_layout_passes=False),
        scratch_types=[
            pltpu.VMEM((N,), jnp.float32),
            pltpu.VMEM((CH,), jnp.float32),
            pltpu.VMEM((CH,), jnp.int32),
            pltpu.VMEM((CH,), jnp.int32),
            pltpu.VMEM((CH,), jnp.int32),
            pltpu.VMEM((CH,), jnp.int32),
            pltpu.VMEM((CH,), jnp.int32),
            pltpu.VMEM((CH,), jnp.int32),
            pltpu.VMEM((CH,), jnp.float32),
            pltpu.VMEM((CH,), jnp.float32),
            pltpu.VMEM((CH,), jnp.float32),
            pltpu.VMEM((CH, F), jnp.float32),
            pltpu.VMEM((CH, F), jnp.float32),
            pltpu.VMEM((CH, F), jnp.float32),
            pltpu.SemaphoreType.DMA,
            pltpu.SemaphoreType.DMA,
            pltpu.SemaphoreType.DMA,
            pltpu.SemaphoreType.DMA,
            pltpu.SemaphoreType.DMA,
            pltpu.SemaphoreType.DMA,
            pltpu.VMEM_SHARED((N, F), jnp.float32),
        ],
    )(src, dst, edge_weight, xw, dinv1, zeros2)


def _dinv_body(deg_ref, o_ref):
    o_ref[...] = lax.rsqrt(deg_ref[0] + deg_ref[1] + 1.0)


def _dinv_call(deg3):
    return pl.pallas_call(
        _dinv_body,
        out_shape=jax.ShapeDtypeStruct((80, 125), jnp.float32),
    )(deg3)


def _post_body(parts_ref, dinv_ref, xw_ref, wlinT_ref, blin_ref, y_ref):
    dv = dinv_ref[...]
    o = parts_ref[0] + parts_ref[1] + dv * dv * xw_ref[...]
    h = jnp.maximum(o, 0.0)
    y_ref[...] = jnp.dot(h, wlinT_ref[...],
                         preferred_element_type=jnp.float32) + blin_ref[...]


def _post_call(outp, dinv2, xw, wlinT, blin2):
    return pl.pallas_call(
        _post_body,
        out_shape=jax.ShapeDtypeStruct((N, 1), jnp.float32),
    )(outp, dinv2, xw, wlinT, blin2)


def kernel(x, edge_index, edge_weight, p, W_ih, W_hh, b_ih, b_hh, W0, W_lin, b_lin):
    xT = jnp.pad(x.T, ((0, 0), (0, NP - N)))
    p2 = p.reshape(1, F)
    wihT = W_ih.T
    whhT = W_hh.T
    bih2 = b_ih.reshape(1, 3 * F)
    bhh2 = b_hh.reshape(1, 3 * F)
    zeros1 = jnp.zeros((N,), jnp.float32)
    zeros2 = jnp.zeros((CH, F), jnp.float32)
    src = edge_index[0]
    dst = edge_index[1]

    deg_parts = _deg_call(dst, edge_weight, zeros1)
    xw = _pre_call(xT, x, p2, wihT, whhT, bih2, bhh2, W0)
    dinv = _dinv_call(deg_parts.reshape(NC, 80, 125))
    outp = _agg_call(src, dst, edge_weight, xw, dinv.reshape(N), zeros2)
    y = _post_call(outp, dinv.reshape(N, 1), xw, W_lin.T, b_lin.reshape(1, 1))
    return y
